# Initial kernel scaffold; baseline (speedup 1.0000x reference)
import jax
import jax.numpy as jnp
from jax.experimental import pallas as pl


def kernel(x, gate_weight):
    raise NotImplementedError("write your pallas kernel here")


# speedup vs baseline: 1.0047x; 1.0047x over previous
import functools

import jax
import jax.numpy as jnp
from jax.experimental import pallas as pl
from jax.experimental.pallas import tpu as pltpu

NUM_SELECTS = 8
BT = 512


def _gate_body(x_ref, wt_ref, idx_ref, score_ref):
    logits = jnp.dot(x_ref[...], wt_ref[...], preferred_element_type=jnp.float32)
    bt, e = logits.shape
    col = jax.lax.broadcasted_iota(jnp.int32, (bt, e), 1)
    neg_inf = jnp.float32(-jnp.inf)

    work = logits
    vals = []
    idxs = []
    for _ in range(NUM_SELECTS):
        m = jnp.max(work, axis=1, keepdims=True)
        hit = work == m
        idx = jnp.min(jnp.where(hit, col, e), axis=1, keepdims=True)
        vals.append(m)
        idxs.append(idx)
        work = jnp.where(col == idx, neg_inf, work)

    v = jnp.concatenate(vals, axis=1)
    i = jnp.concatenate(idxs, axis=1)
    ex = jnp.exp(v - v[:, 0:1])
    s = ex / jnp.sum(ex, axis=1, keepdims=True)
    idx_ref[...] = i
    score_ref[...] = s


@jax.jit
def kernel(x, gate_weight):
    t, d = x.shape
    e = gate_weight.shape[0]
    wt = gate_weight.T
    grid = (t // BT,)
    idx, score = pl.pallas_call(
        _gate_body,
        grid=grid,
        in_specs=[
            pl.BlockSpec((BT, d), lambda i: (i, 0)),
            pl.BlockSpec((d, e), lambda i: (0, 0)),
        ],
        out_specs=[
            pl.BlockSpec((BT, NUM_SELECTS), lambda i: (i, 0)),
            pl.BlockSpec((BT, NUM_SELECTS), lambda i: (i, 0)),
        ],
        out_shape=[
            jax.ShapeDtypeStruct((t, NUM_SELECTS), jnp.int32),
            jax.ShapeDtypeStruct((t, NUM_SELECTS), jnp.float32),
        ],
        compiler_params=pltpu.CompilerParams(
            dimension_semantics=("arbitrary",),
        ),
    )(x, wt)
    return idx, score


# speedup vs baseline: 1.0205x; 1.0158x over previous
import jax
import jax.numpy as jnp
from jax.experimental import pallas as pl
from jax.experimental.pallas import tpu as pltpu

NUM_SELECTS = 8
BT = 512


def _gate_body(x_ref, wt_ref, idx_ref, score_ref):
    x_bf = x_ref[...].astype(jnp.bfloat16)
    logits = jnp.dot(x_bf, wt_ref[...], preferred_element_type=jnp.float32)

    bt, e = logits.shape
    col = jax.lax.broadcasted_iota(jnp.int32, (bt, e), 1)
    neg_inf = jnp.float32(-jnp.inf)

    work = logits
    vals = []
    idxs = []
    for _ in range(NUM_SELECTS):
        m = jnp.max(work, axis=1, keepdims=True)
        hit = work == m
        idx = jnp.min(jnp.where(hit, col, e), axis=1, keepdims=True)
        vals.append(m)
        idxs.append(idx)
        work = jnp.where(col == idx, neg_inf, work)

    v = jnp.concatenate(vals, axis=1)
    i = jnp.concatenate(idxs, axis=1)
    ex = jnp.exp(v - v[:, 0:1])
    s = ex / jnp.sum(ex, axis=1, keepdims=True)
    idx_ref[...] = i
    score_ref[...] = s


@jax.jit
def kernel(x, gate_weight):
    t, d = x.shape
    e = gate_weight.shape[0]
    wt = gate_weight.T.astype(jnp.bfloat16)
    grid = (t // BT,)
    idx, score = pl.pallas_call(
        _gate_body,
        grid=grid,
        in_specs=[
            pl.BlockSpec((BT, d), lambda i: (i, 0)),
            pl.BlockSpec((d, e), lambda i: (0, 0)),
        ],
        out_specs=[
            pl.BlockSpec((BT, NUM_SELECTS), lambda i: (i, 0)),
            pl.BlockSpec((BT, NUM_SELECTS), lambda i: (i, 0)),
        ],
        out_shape=[
            jax.ShapeDtypeStruct((t, NUM_SELECTS), jnp.int32),
            jax.ShapeDtypeStruct((t, NUM_SELECTS), jnp.float32),
        ],
        compiler_params=pltpu.CompilerParams(
            dimension_semantics=("arbitrary",),
        ),
    )(x, wt)
    return idx, score
